# baseline (speedup 1.0000x reference)
import functools
import jax
import jax.numpy as jnp
from jax import lax
from jax.experimental import pallas as pl
from jax.experimental.pallas import tpu as pltpu
from jax.experimental.pallas import tpu_sc as plsc

_B, _N = 4, 4096
_L = 16
_QBLK = 8
_NW = 32


def _sc_symloss(xr_hbm, yr_hbm, zr_hbm, xe_hbm, ye_hbm, ze_hbm,
                out_hbm, xv, yv, zv, nv, ov):
    wid = lax.axis_index("s") * 2 + lax.axis_index("c")
    rows_per_w = (_B * _N) // _NW
    wpb = _NW // _B
    b = wid // wpb
    q0 = (wid % wpb) * rows_per_w

    pltpu.sync_copy(xe_hbm.at[b], xv)
    pltpu.sync_copy(ye_hbm.at[b], yv)
    pltpu.sync_copy(ze_hbm.at[b], zv)

    def norm_body(c, carry):
        xe = xv[pl.ds(c * _L, _L)]
        ye = yv[pl.ds(c * _L, _L)]
        ze = zv[pl.ds(c * _L, _L)]
        nv[pl.ds(c * _L, _L)] = (xe * xe + ye * ye) + ze * ze
        return carry

    lax.fori_loop(0, _N // _L, norm_body, jnp.float32(0.0))

    pltpu.sync_copy(xr_hbm.at[b], xv)
    pltpu.sync_copy(yr_hbm.at[b], yv)
    pltpu.sync_copy(zr_hbm.at[b], zv)

    inf = jnp.full((_L,), jnp.inf, dtype=jnp.float32)

    def group_body(g, total):
        base = q0 + g * _L
        qxv = xv[pl.ds(base, _L)]
        qyv = yv[pl.ds(base, _L)]
        qzv = zv[pl.ds(base, _L)]
        qnv = nv[pl.ds(base, _L)]
        for h in range(_L // _QBLK):
            qxs, qys, qzs, qns = [], [], [], []
            for j in range(_QBLK):
                lane = h * _QBLK + j
                qxs.append(jnp.full((_L,), 2.0 * qxv[lane]))
                qys.append(jnp.full((_L,), -2.0 * qyv[lane]))
                qzs.append(jnp.full((_L,), -2.0 * qzv[lane]))
                qns.append(jnp.full((_L,), qnv[lane]))

            def chunk_body(c, accs):
                xm = xv[pl.ds(c * _L, _L)]
                ym = yv[pl.ds(c * _L, _L)]
                zm = zv[pl.ds(c * _L, _L)]
                nm = nv[pl.ds(c * _L, _L)]
                out = []
                for j in range(_QBLK):
                    t = (qxs[j] * xm + qys[j] * ym) + qzs[j] * zm
                    d = (nm + qns[j]) + t
                    out.append(jnp.minimum(accs[j], d))
                return tuple(out)

            accs = lax.fori_loop(0, _N // _L, chunk_body,
                                 tuple([inf] * _QBLK))
            for j in range(_QBLK):
                m = accs[j]
                for stride in (8, 4, 2, 1):
                    idx = jnp.bitwise_xor(
                        lax.iota(jnp.int32, _L),
                        jnp.int32(stride))
                    perm = lax.gather(
                        m, idx[:, None],
                        lax.GatherDimensionNumbers(
                            offset_dims=(), collapsed_slice_dims=(0,),
                            start_index_map=(0,)),
                        slice_sizes=(1,),
                        mode=lax.GatherScatterMode.PROMISE_IN_BOUNDS)
                    m = jnp.minimum(m, perm)
                total = total + m[0]
        return total

    total = lax.fori_loop(0, rows_per_w // _L, group_body,
                          jnp.float32(0.0))
    ov[...] = jnp.full((_L,), total, dtype=jnp.float32)
    pltpu.sync_copy(ov, out_hbm.at[wid])


def _round_bf16(x):
    u = lax.bitcast_convert_type(x, jnp.uint32)
    u = (u + jnp.uint32(0x7FFF) + ((u >> 16) & jnp.uint32(1))) \
        & jnp.uint32(0xFFFF0000)
    return lax.bitcast_convert_type(u, jnp.float32)


def sc_partial_sums(xyz):
    B, N, _ = xyz.shape
    xr = _round_bf16(xyz[:, :, 0])
    yr = _round_bf16(xyz[:, :, 1])
    zr = _round_bf16(xyz[:, :, 2])
    xe = xyz[:, :, 0]
    ye = xyz[:, :, 1]
    ze = xyz[:, :, 2]
    mesh = plsc.VectorSubcoreMesh(core_axis_name="c", subcore_axis_name="s")
    run = functools.partial(
        pl.kernel,
        mesh=mesh,
        out_type=jax.ShapeDtypeStruct((_NW, _L), jnp.float32),
        scratch_types=[
            pltpu.VMEM((N,), jnp.float32),
            pltpu.VMEM((N,), jnp.float32),
            pltpu.VMEM((N,), jnp.float32),
            pltpu.VMEM((N,), jnp.float32),
            pltpu.VMEM((_L,), jnp.float32),
        ],
    )(_sc_symloss)
    return run(xr, yr, zr, xe, ye, ze)


def kernel(xyz):
    B, N, _ = xyz.shape
    ps = sc_partial_sums(xyz)
    return jnp.sum(ps[:, 0]) * (2.0 / (B * N))


# speedup vs baseline: 3.2914x; 3.2914x over previous
import functools
import jax
import jax.numpy as jnp
from jax import lax
from jax.experimental import pallas as pl
from jax.experimental.pallas import tpu as pltpu
from jax.experimental.pallas import tpu_sc as plsc

_B, _N = 4, 4096
_R_SC = 512
_TILE = 1792
_L = 16
_QBLK = 8
_NW = 32


def _tc_body(q_ref, t_ref, out_ref):
    b = pl.program_id(0)
    i = pl.program_id(1)
    q = q_ref[0]
    qx = q[:, 0:1]
    qy = q[:, 1:2]
    qz = q[:, 2:3]
    t = t_ref[0]
    qn = (qx * qx + qy * qy) + qz * qz
    tn = jnp.sum(t * t, axis=1, keepdims=True)
    tn_hi = tn.astype(jnp.bfloat16).astype(jnp.float32)
    tn_lo = tn - tn_hi
    ones = jnp.ones_like(qx)
    a_aug = jnp.concatenate(
        [2.0 * qx, -2.0 * qy, -2.0 * qz, ones, ones], axis=1)
    t_aug = jnp.concatenate([t, tn_hi, tn_lo], axis=1)
    e = lax.dot_general(a_aug.astype(jnp.bfloat16),
                        t_aug.astype(jnp.bfloat16),
                        (((1,), (1,)), ((), ())),
                        preferred_element_type=jnp.float32)
    s = jnp.sum(qn) + jnp.sum(jnp.min(e, axis=1))

    @pl.when((b == 0) & (i == 0))
    def _init():
        out_ref[0, 0] = 0.0

    out_ref[0, 0] += s


def _tc_partial_sum(xyz):
    B, N, _ = xyz.shape
    q_rows = xyz[:, _R_SC:, :]
    return pl.pallas_call(
        _tc_body,
        grid=(B, (N - _R_SC) // _TILE),
        in_specs=[
            pl.BlockSpec((1, _TILE, 3), lambda b, i: (b, i, 0)),
            pl.BlockSpec((1, N, 3), lambda b, i: (b, 0, 0)),
        ],
        out_specs=pl.BlockSpec((1, 1), lambda b, i: (0, 0),
                               memory_space=pltpu.SMEM),
        out_shape=jax.ShapeDtypeStruct((1, 1), jnp.float32),
    )(q_rows, xyz)


def _sc_symloss(xr_hbm, yr_hbm, zr_hbm, xe_hbm, ye_hbm, ze_hbm,
                out_hbm, xv, yv, zv, nv, ov):
    wid = lax.axis_index("s") * 2 + lax.axis_index("c")
    rows_per_w = (_B * _R_SC) // _NW
    wpb = _NW // _B
    b = wid // wpb
    q0 = (wid % wpb) * rows_per_w

    pltpu.sync_copy(xe_hbm.at[b], xv)
    pltpu.sync_copy(ye_hbm.at[b], yv)
    pltpu.sync_copy(ze_hbm.at[b], zv)

    def norm_body(c, carry):
        xe = xv[pl.ds(c * _L, _L)]
        ye = yv[pl.ds(c * _L, _L)]
        ze = zv[pl.ds(c * _L, _L)]
        nv[pl.ds(c * _L, _L)] = (xe * xe + ye * ye) + ze * ze
        return carry

    lax.fori_loop(0, _N // _L, norm_body, jnp.float32(0.0))

    pltpu.sync_copy(xr_hbm.at[b], xv)
    pltpu.sync_copy(yr_hbm.at[b], yv)
    pltpu.sync_copy(zr_hbm.at[b], zv)

    inf = jnp.full((_L,), jnp.inf, dtype=jnp.float32)

    def group_body(g, total):
        base = q0 + g * _L
        qxv = xv[pl.ds(base, _L)]
        qyv = yv[pl.ds(base, _L)]
        qzv = zv[pl.ds(base, _L)]
        qnv = nv[pl.ds(base, _L)]
        for h in range(_L // _QBLK):
            qxs, qys, qzs, qns = [], [], [], []
            for j in range(_QBLK):
                lane = h * _QBLK + j
                qxs.append(jnp.full((_L,), 2.0 * qxv[lane]))
                qys.append(jnp.full((_L,), -2.0 * qyv[lane]))
                qzs.append(jnp.full((_L,), -2.0 * qzv[lane]))
                qns.append(jnp.full((_L,), qnv[lane]))

            def chunk_body(c, accs):
                xm = xv[pl.ds(c * _L, _L)]
                ym = yv[pl.ds(c * _L, _L)]
                zm = zv[pl.ds(c * _L, _L)]
                nm = nv[pl.ds(c * _L, _L)]
                out = []
                for j in range(_QBLK):
                    t = (qxs[j] * xm + qys[j] * ym) + qzs[j] * zm
                    d = (nm + qns[j]) + t
                    out.append(jnp.minimum(accs[j], d))
                return tuple(out)

            accs = lax.fori_loop(0, _N // _L, chunk_body,
                                 tuple([inf] * _QBLK))
            for j in range(_QBLK):
                m = accs[j]
                for stride in (8, 4, 2, 1):
                    idx = jnp.bitwise_xor(
                        lax.iota(jnp.int32, _L),
                        jnp.int32(stride))
                    perm = lax.gather(
                        m, idx[:, None],
                        lax.GatherDimensionNumbers(
                            offset_dims=(), collapsed_slice_dims=(0,),
                            start_index_map=(0,)),
                        slice_sizes=(1,),
                        mode=lax.GatherScatterMode.PROMISE_IN_BOUNDS)
                    m = jnp.minimum(m, perm)
                total = total + m[0]
        return total

    total = lax.fori_loop(0, rows_per_w // _L, group_body,
                          jnp.float32(0.0))
    ov[...] = jnp.full((_L,), total, dtype=jnp.float32)
    pltpu.sync_copy(ov, out_hbm.at[wid])


def _round_bf16(x):
    u = lax.bitcast_convert_type(x, jnp.uint32)
    u = (u + jnp.uint32(0x7FFF) + ((u >> 16) & jnp.uint32(1))) \
        & jnp.uint32(0xFFFF0000)
    return lax.bitcast_convert_type(u, jnp.float32)


def _sc_partial_sums(xyz):
    B, N, _ = xyz.shape
    xr = _round_bf16(xyz[:, :, 0])
    yr = _round_bf16(xyz[:, :, 1])
    zr = _round_bf16(xyz[:, :, 2])
    xe = xyz[:, :, 0]
    ye = xyz[:, :, 1]
    ze = xyz[:, :, 2]
    mesh = plsc.VectorSubcoreMesh(core_axis_name="c", subcore_axis_name="s")
    run = functools.partial(
        pl.kernel,
        mesh=mesh,
        out_type=jax.ShapeDtypeStruct((_NW, _L), jnp.float32),
        scratch_types=[
            pltpu.VMEM((N,), jnp.float32),
            pltpu.VMEM((N,), jnp.float32),
            pltpu.VMEM((N,), jnp.float32),
            pltpu.VMEM((N,), jnp.float32),
            pltpu.VMEM((_L,), jnp.float32),
        ],
    )(_sc_symloss)
    return run(xr, yr, zr, xe, ye, ze)


def kernel(xyz):
    B, N, _ = xyz.shape
    sc = _sc_partial_sums(xyz)
    tc = _tc_partial_sum(xyz)
    total = tc[0, 0] + jnp.sum(sc[:, 0])
    return total * (2.0 / (B * N))


# speedup vs baseline: 3.2971x; 1.0017x over previous
import functools
import jax
import jax.numpy as jnp
from jax import lax
from jax.experimental import pallas as pl
from jax.experimental.pallas import tpu as pltpu
from jax.experimental.pallas import tpu_sc as plsc

_B, _N = 4, 4096
_R_SC = 512
_TILE = 1792
_L = 16
_QBLK = 8
_NW = 32


def _tc_body(q_ref, t_ref, out_ref):
    b = pl.program_id(0)
    i = pl.program_id(1)
    q = q_ref[0]
    qx = q[:, 0:1]
    qy = q[:, 1:2]
    qz = q[:, 2:3]
    t = t_ref[0]
    qn = (qx * qx + qy * qy) + qz * qz
    tn = jnp.sum(t * t, axis=1, keepdims=True)
    tn_hi = tn.astype(jnp.bfloat16).astype(jnp.float32)
    tn_lo = tn - tn_hi
    ones = jnp.ones_like(qx)
    a_aug = jnp.concatenate(
        [2.0 * qx, -2.0 * qy, -2.0 * qz, ones, ones], axis=1)
    t_aug = jnp.concatenate([t, tn_hi, tn_lo], axis=1)
    e = lax.dot_general(a_aug.astype(jnp.bfloat16),
                        t_aug.astype(jnp.bfloat16),
                        (((1,), (1,)), ((), ())),
                        preferred_element_type=jnp.float32)
    s = jnp.sum(qn) + jnp.sum(jnp.min(e, axis=1))

    @pl.when((b == 0) & (i == 0))
    def _init():
        out_ref[0, 0] = 0.0

    out_ref[0, 0] += s


def _tc_partial_sum(xyz):
    B, N, _ = xyz.shape
    q_rows = xyz[:, _R_SC:, :]
    return pl.pallas_call(
        _tc_body,
        grid=(B, (N - _R_SC) // _TILE),
        in_specs=[
            pl.BlockSpec((1, _TILE, 3), lambda b, i: (b, i, 0)),
            pl.BlockSpec((1, N, 3), lambda b, i: (b, 0, 0)),
        ],
        out_specs=pl.BlockSpec((1, 1), lambda b, i: (0, 0),
                               memory_space=pltpu.SMEM),
        out_shape=jax.ShapeDtypeStruct((1, 1), jnp.float32),
    )(q_rows, xyz)


def _sc_symloss(xr_hbm, yr_hbm, zr_hbm, xe_hbm, ye_hbm, ze_hbm,
                out_hbm, xv, yv, zv, nv, ov):
    wid = lax.axis_index("s") * 2 + lax.axis_index("c")
    rows_per_w = (_B * _R_SC) // _NW
    wpb = _NW // _B
    b = wid // wpb
    q0 = (wid % wpb) * rows_per_w

    pltpu.sync_copy(xe_hbm.at[b], xv)
    pltpu.sync_copy(ye_hbm.at[b], yv)
    pltpu.sync_copy(ze_hbm.at[b], zv)

    def norm_body(c, carry):
        xe = xv[pl.ds(c * _L, _L)]
        ye = yv[pl.ds(c * _L, _L)]
        ze = zv[pl.ds(c * _L, _L)]
        nv[pl.ds(c * _L, _L)] = (xe * xe + ye * ye) + ze * ze
        return carry

    lax.fori_loop(0, _N // _L, norm_body, jnp.float32(0.0))

    pltpu.sync_copy(xr_hbm.at[b], xv)
    pltpu.sync_copy(yr_hbm.at[b], yv)
    pltpu.sync_copy(zr_hbm.at[b], zv)

    inf = jnp.full((_L,), jnp.inf, dtype=jnp.float32)

    def group_body(g, total):
        base = q0 + g * _L
        qxv = xv[pl.ds(base, _L)]
        qyv = yv[pl.ds(base, _L)]
        qzv = zv[pl.ds(base, _L)]
        qnv = nv[pl.ds(base, _L)]
        for h in range(_L // _QBLK):
            qxs, qys, qzs, qns = [], [], [], []
            for j in range(_QBLK):
                lane = h * _QBLK + j
                qxs.append(jnp.full((_L,), 2.0 * qxv[lane]))
                qys.append(jnp.full((_L,), -2.0 * qyv[lane]))
                qzs.append(jnp.full((_L,), -2.0 * qzv[lane]))
                qns.append(jnp.full((_L,), qnv[lane]))

            def chunk_body(c, accs):
                xm = xv[pl.ds(c * _L, _L)]
                ym = yv[pl.ds(c * _L, _L)]
                zm = zv[pl.ds(c * _L, _L)]
                nm = nv[pl.ds(c * _L, _L)]
                out = []
                for j in range(_QBLK):
                    t = (qxs[j] * xm + qys[j] * ym) + qzs[j] * zm
                    d = (nm + qns[j]) + t
                    out.append(jnp.minimum(accs[j], d))
                return tuple(out)

            accs = lax.fori_loop(0, _N // _L, chunk_body,
                                 tuple([inf] * _QBLK))
            for j in range(_QBLK):
                m = accs[j]
                for stride in (8, 4, 2, 1):
                    idx = jnp.bitwise_xor(
                        lax.iota(jnp.int32, _L),
                        jnp.int32(stride))
                    perm = lax.gather(
                        m, idx[:, None],
                        lax.GatherDimensionNumbers(
                            offset_dims=(), collapsed_slice_dims=(0,),
                            start_index_map=(0,)),
                        slice_sizes=(1,),
                        mode=lax.GatherScatterMode.PROMISE_IN_BOUNDS)
                    m = jnp.minimum(m, perm)
                total = total + m[0]
        return total

    total = lax.fori_loop(0, rows_per_w // _L, group_body,
                          jnp.float32(0.0))
    ov[...] = jnp.full((_L,), total, dtype=jnp.float32)
    pltpu.sync_copy(ov, out_hbm.at[wid])


def _round_bf16(x):
    u = lax.bitcast_convert_type(x, jnp.uint32)
    u = (u + jnp.uint32(0x7FFF) + ((u >> 16) & jnp.uint32(1))) \
        & jnp.uint32(0xFFFF0000)
    return lax.bitcast_convert_type(u, jnp.float32)


def _sc_partial_sums(xyz):
    B, N, _ = xyz.shape
    xr = _round_bf16(xyz[:, :, 0])
    yr = _round_bf16(xyz[:, :, 1])
    zr = _round_bf16(xyz[:, :, 2])
    xe = xyz[:, :, 0]
    ye = xyz[:, :, 1]
    ze = xyz[:, :, 2]
    mesh = plsc.VectorSubcoreMesh(core_axis_name="c", subcore_axis_name="s")
    run = functools.partial(
        pl.kernel,
        mesh=mesh,
        out_type=jax.ShapeDtypeStruct((_NW, _L), jnp.float32),
        scratch_types=[
            pltpu.VMEM((N,), jnp.float32),
            pltpu.VMEM((N,), jnp.float32),
            pltpu.VMEM((N,), jnp.float32),
            pltpu.VMEM((N,), jnp.float32),
            pltpu.VMEM((_L,), jnp.float32),
        ],
    )(_sc_symloss)
    return run(xr, yr, zr, xe, ye, ze)


def kernel(xyz):
    B, N, _ = xyz.shape
    tc = _tc_partial_sum(xyz)
    sc = _sc_partial_sums(xyz)
    total = tc[0, 0] + jnp.sum(sc[:, 0])
    return total * (2.0 / (B * N))


# speedup vs baseline: 4.8344x; 1.4663x over previous
import jax
import jax.numpy as jnp
from jax import lax
from jax.experimental import pallas as pl
from jax.experimental.pallas import tpu as pltpu

_B, _N = 4, 4096
_TILE = 2048


def _symloss_body(q_ref, t_ref, out_ref):
    b = pl.program_id(0)
    i = pl.program_id(1)
    q = q_ref[0]
    qx = q[:, 0:1]
    qy = q[:, 1:2]
    qz = q[:, 2:3]
    t = t_ref[0]
    qn = (qx * qx + qy * qy) + qz * qz
    tn = jnp.sum(t * t, axis=1, keepdims=True)
    tn_hi = tn.astype(jnp.bfloat16).astype(jnp.float32)
    tn_lo = tn - tn_hi
    ones = jnp.ones_like(qx)
    a_aug = jnp.concatenate(
        [2.0 * qx, -2.0 * qy, -2.0 * qz, ones, ones], axis=1)
    t_aug = jnp.concatenate([t, tn_hi, tn_lo], axis=1)
    e = lax.dot_general(a_aug.astype(jnp.bfloat16),
                        t_aug.astype(jnp.bfloat16),
                        (((1,), (1,)), ((), ())),
                        preferred_element_type=jnp.float32)
    s = jnp.sum(qn) + jnp.sum(jnp.min(e, axis=1))

    @pl.when((b == 0) & (i == 0))
    def _init():
        out_ref[0, 0] = 0.0

    out_ref[0, 0] += s


def kernel(xyz):
    B, N, _ = xyz.shape
    total = pl.pallas_call(
        _symloss_body,
        grid=(B, N // _TILE),
        in_specs=[
            pl.BlockSpec((1, _TILE, 3), lambda b, i: (b, i, 0)),
            pl.BlockSpec((1, N, 3), lambda b, i: (b, 0, 0)),
        ],
        out_specs=pl.BlockSpec((1, 1), lambda b, i: (0, 0),
                               memory_space=pltpu.SMEM),
        out_shape=jax.ShapeDtypeStruct((1, 1), jnp.float32),
    )(xyz, xyz)
    return total[0, 0] * (2.0 / (B * N))
